# baseline (speedup 1.0000x reference)
import functools

import jax
import jax.numpy as jnp
from jax import lax
from jax.experimental import pallas as pl
from jax.experimental.pallas import tpu as pltpu
from jax.experimental.pallas import tpu_sc as plsc

N_NODES = 10000
N_EDGES = 320000
DIM = 128
ALPHA = 0.1
BETA = 1.0

NC = 2
NS = 16
NW = NC * NS
CHUNK = 128
NCHUNK = 80
SCHUNK = 40
EPAD = NW * NCHUNK * CHUNK
NPAD = 10240
PAD_DST = N_NODES
ROWS_PER_TILE = NPAD // NS
ZCHUNK = 80
WCHUNK = 128
LANES = 16

_MESH = plsc.VectorSubcoreMesh(
    core_axis_name="c", subcore_axis_name="s", num_cores=NC, num_subcores=NS
)


@functools.partial(
    pl.kernel,
    out_type=jax.ShapeDtypeStruct((NC, NPAD, DIM), jnp.float32),
    mesh=_MESH,
    scratch_types=[
        pltpu.VMEM((SCHUNK, CHUNK), jnp.int32),
        pltpu.VMEM((SCHUNK, CHUNK), jnp.int32),
        pltpu.VMEM((CHUNK, DIM), jnp.float32),
        pltpu.VMEM_SHARED((NPAD, DIM), jnp.float32),
        pltpu.SemaphoreType.DMA,
    ],
)
def _spmm_sc(x_hbm, src_hbm, dst_hbm, part_hbm, src_v, dst_v, rows_v, acc_sh,
             sem):
    c = lax.axis_index("c")
    s = lax.axis_index("s")
    wid = s * NC + c

    def _zfill(t, carry):
        i = t // (DIM // LANES)
        k = t % (DIM // LANES)
        rows_v[i, pl.ds(k * LANES, LANES)] = jnp.zeros((LANES,), jnp.float32)
        return carry

    lax.fori_loop(0, ZCHUNK * (DIM // LANES), _zfill, 0)

    def _zcopy(t, carry):
        off = pl.multiple_of(s * ROWS_PER_TILE + t * ZCHUNK, 8)
        pltpu.sync_copy(rows_v.at[pl.ds(0, ZCHUNK)], acc_sh.at[pl.ds(off, ZCHUNK)])
        return carry

    lax.fori_loop(0, ROWS_PER_TILE // ZCHUNK, _zcopy, 0)
    plsc.subcore_barrier()

    for sbase in (0, SCHUNK):
        pltpu.sync_copy(src_hbm.at[wid, pl.ds(sbase, SCHUNK)], src_v)
        pltpu.sync_copy(dst_hbm.at[wid, pl.ds(sbase, SCHUNK)], dst_v)

        def _edge_chunk(j, carry):
            pltpu.async_copy(x_hbm.at[src_v.at[j]], rows_v, sem).wait()
            pltpu.sync_copy(rows_v, acc_sh.at[dst_v.at[j]], add=True)
            return carry

        lax.fori_loop(0, SCHUNK, _edge_chunk, 0)
    plsc.subcore_barrier()

    def _wback(t, carry):
        off = pl.multiple_of(s * ROWS_PER_TILE + t * WCHUNK, 8)
        pltpu.sync_copy(acc_sh.at[pl.ds(off, WCHUNK)],
                        part_hbm.at[c, pl.ds(off, WCHUNK)])
        return carry

    lax.fori_loop(0, ROWS_PER_TILE // WCHUNK, _wback, 0)


RBLK = 2000


def _mix_mm_tc(part_ref, init_ref, w_ref, b_ref, out_ref):
    h = (1.0 - ALPHA) * (part_ref[0] + part_ref[1]) + ALPHA * init_ref[...]
    mm = lax.dot_general(h, w_ref[...], (((1,), (1,)), ((), ())),
                         preferred_element_type=jnp.float32)
    out_ref[...] = BETA * (mm + b_ref[...]) + (1.0 - BETA) * h


def kernel(x, init_x, edge_index, W, b):
    npad = EPAD - N_EDGES
    src = jnp.concatenate(
        [edge_index[0], jnp.zeros((npad,), jnp.int32)]
    ).reshape(NW, NCHUNK, CHUNK)
    pad_dst = PAD_DST + jnp.arange(npad, dtype=jnp.int32) % (NPAD - N_NODES)
    dst = jnp.concatenate(
        [edge_index[1], pad_dst]
    ).reshape(NW, NCHUNK, CHUNK)
    part = _spmm_sc(x, src, dst)
    out = pl.pallas_call(
        _mix_mm_tc,
        grid=(N_NODES // RBLK,),
        in_specs=[
            pl.BlockSpec((NC, RBLK, DIM), lambda i: (0, i, 0)),
            pl.BlockSpec((RBLK, DIM), lambda i: (i, 0)),
            pl.BlockSpec((DIM, DIM), lambda i: (0, 0)),
            pl.BlockSpec((1, DIM), lambda i: (0, 0)),
        ],
        out_specs=pl.BlockSpec((RBLK, DIM), lambda i: (i, 0)),
        out_shape=jax.ShapeDtypeStruct((N_NODES, DIM), jnp.float32),
    )(part, init_x, W, b.reshape(1, DIM))
    return out


# speedup vs baseline: 2.2509x; 2.2509x over previous
import functools

import jax
import jax.numpy as jnp
from jax import lax
from jax.experimental import pallas as pl
from jax.experimental.pallas import tpu as pltpu
from jax.experimental.pallas import tpu_sc as plsc

N_NODES = 10000
N_EDGES = 320000
DIM = 128
ALPHA = 0.1
BETA = 1.0

NC = 2
NS = 16
NW = NC * NS
CHUNK = 128
NCHUNK = 80
SCHUNK = 40
EPAD = NW * NCHUNK * CHUNK
NPAD = 10240
PAD_DST = N_NODES
ROWS_PER_TILE = NPAD // NS
ZCHUNK = 80
WCHUNK = 128
LANES = 16

_MESH = plsc.VectorSubcoreMesh(
    core_axis_name="c", subcore_axis_name="s", num_cores=NC, num_subcores=NS
)


@functools.partial(
    pl.kernel,
    out_type=jax.ShapeDtypeStruct((NC, NPAD, DIM), jnp.float32),
    mesh=_MESH,
    scratch_types=[
        pltpu.VMEM((SCHUNK, CHUNK), jnp.int32),
        pltpu.VMEM((SCHUNK, CHUNK), jnp.int32),
        pltpu.VMEM((CHUNK, DIM), jnp.float32),
        pltpu.VMEM_SHARED((NPAD, DIM), jnp.float32),
        pltpu.SemaphoreType.DMA,
    ],
)
def _spmm_sc(x_hbm, src_hbm, dst_hbm, part_hbm, src_v, dst_v, rows_v, acc_sh,
             sem):
    c = lax.axis_index("c")
    s = lax.axis_index("s")
    wid = s * NC + c

    def _zfill(t, carry):
        i = t // (DIM // LANES)
        k = t % (DIM // LANES)
        rows_v[i, pl.ds(k * LANES, LANES)] = jnp.zeros((LANES,), jnp.float32)
        return carry

    lax.fori_loop(0, ZCHUNK * (DIM // LANES), _zfill, 0)

    def _zcopy(t, carry):
        off = pl.multiple_of(s * ROWS_PER_TILE + t * ZCHUNK, 8)
        pltpu.sync_copy(rows_v.at[pl.ds(0, ZCHUNK)], acc_sh.at[pl.ds(off, ZCHUNK)])
        return carry

    lax.fori_loop(0, ROWS_PER_TILE // ZCHUNK, _zcopy, 0)
    plsc.subcore_barrier()

    for sbase in (0, SCHUNK):
        pltpu.sync_copy(src_hbm.at[wid, pl.ds(sbase, SCHUNK)], src_v)
        pltpu.sync_copy(dst_hbm.at[wid, pl.ds(sbase, SCHUNK)], dst_v)

        def _edge_chunk(j, carry):
            pltpu.async_copy(x_hbm.at[src_v.at[j]], rows_v, sem).wait()
            pltpu.sync_copy(rows_v, acc_sh.at[dst_v.at[j]], add=True)
            return carry

        lax.fori_loop(0, SCHUNK, _edge_chunk, 0)
    plsc.subcore_barrier()

    def _wback(t, carry):
        off = pl.multiple_of(s * ROWS_PER_TILE + t * WCHUNK, 8)
        pltpu.sync_copy(acc_sh.at[pl.ds(off, WCHUNK)],
                        part_hbm.at[c, pl.ds(off, WCHUNK)])
        return carry

    lax.fori_loop(0, ROWS_PER_TILE // WCHUNK, _wback, 0)


RBLK = 2000


def _mix_mm_tc(part_ref, init_ref, w_ref, b_ref, out_ref):
    h = (1.0 - ALPHA) * (part_ref[0] + part_ref[1]) + ALPHA * init_ref[...]
    mm = lax.dot_general(h, w_ref[...], (((1,), (1,)), ((), ())),
                         preferred_element_type=jnp.float32)
    out_ref[...] = BETA * (mm + b_ref[...]) + (1.0 - BETA) * h


def kernel(x, init_x, edge_index, W, b):
    npad = EPAD - N_EDGES
    pad_src = jnp.arange(npad, dtype=jnp.int32) % N_NODES
    src = jnp.concatenate(
        [edge_index[0], pad_src]
    ).reshape(NW, NCHUNK, CHUNK)
    pad_dst = PAD_DST + jnp.arange(npad, dtype=jnp.int32) % (NPAD - N_NODES)
    dst = jnp.concatenate(
        [edge_index[1], pad_dst]
    ).reshape(NW, NCHUNK, CHUNK)
    part = _spmm_sc(x, src, dst)
    out = pl.pallas_call(
        _mix_mm_tc,
        grid=(N_NODES // RBLK,),
        in_specs=[
            pl.BlockSpec((NC, RBLK, DIM), lambda i: (0, i, 0)),
            pl.BlockSpec((RBLK, DIM), lambda i: (i, 0)),
            pl.BlockSpec((DIM, DIM), lambda i: (0, 0)),
            pl.BlockSpec((1, DIM), lambda i: (0, 0)),
        ],
        out_specs=pl.BlockSpec((RBLK, DIM), lambda i: (i, 0)),
        out_shape=jax.ShapeDtypeStruct((N_NODES, DIM), jnp.float32),
    )(part, init_x, W, b.reshape(1, DIM))
    return out


# speedup vs baseline: 2.6425x; 1.1740x over previous
import functools

import jax
import jax.numpy as jnp
from jax import lax
from jax.experimental import pallas as pl
from jax.experimental.pallas import tpu as pltpu
from jax.experimental.pallas import tpu_sc as plsc

N_NODES = 10000
N_EDGES = 320000
DIM = 128
ALPHA = 0.1
BETA = 1.0

NC = 2
NS = 16
NW = NC * NS
CHUNK = 128
NCHUNK = 80
SCHUNK = 8
EPAD = NW * NCHUNK * CHUNK
NPAD = 10240
PAD_DST = N_NODES
ROWS_PER_TILE = NPAD // NS
ZCHUNK = 80
WCHUNK = 128
LANES = 16

_MESH = plsc.VectorSubcoreMesh(
    core_axis_name="c", subcore_axis_name="s", num_cores=NC, num_subcores=NS
)


@functools.partial(
    pl.kernel,
    out_type=jax.ShapeDtypeStruct((NC, NPAD, DIM), jnp.float32),
    mesh=_MESH,
    scratch_types=[
        pltpu.VMEM((SCHUNK, CHUNK), jnp.int32),
        pltpu.VMEM((SCHUNK, CHUNK), jnp.int32),
        pltpu.VMEM((2, CHUNK, DIM), jnp.float32),
        pltpu.VMEM_SHARED((NPAD, DIM), jnp.float32),
        pltpu.SemaphoreType.DMA,
        pltpu.SemaphoreType.DMA,
        pltpu.SemaphoreType.DMA,
        pltpu.SemaphoreType.DMA,
    ],
)
def _spmm_sc(x_hbm, src_hbm, dst_hbm, part_hbm, src_v, dst_v, rows_v, acc_sh,
             g0, g1, s0, s1):
    c = lax.axis_index("c")
    s = lax.axis_index("s")
    wid = s * NC + c

    def _zfill(t, carry):
        i = t // (DIM // LANES)
        k = t % (DIM // LANES)
        rows_v[0, i, pl.ds(k * LANES, LANES)] = jnp.zeros((LANES,), jnp.float32)
        return carry

    lax.fori_loop(0, ZCHUNK * (DIM // LANES), _zfill, 0)

    def _zcopy(t, carry):
        off = pl.multiple_of(s * ROWS_PER_TILE + t * ZCHUNK, 8)
        pltpu.sync_copy(rows_v.at[0, pl.ds(0, ZCHUNK)],
                        acc_sh.at[pl.ds(off, ZCHUNK)])
        return carry

    lax.fori_loop(0, ROWS_PER_TILE // ZCHUNK, _zcopy, 0)
    plsc.subcore_barrier()

    npairs = SCHUNK // 2
    for sbase in range(0, NCHUNK, SCHUNK):
        pltpu.sync_copy(src_hbm.at[wid, pl.ds(sbase, SCHUNK)], src_v)
        pltpu.sync_copy(dst_hbm.at[wid, pl.ds(sbase, SCHUNK)], dst_v)
        pltpu.async_copy(x_hbm.at[src_v.at[0]], rows_v.at[0], g0)

        def _pair(p, carry):
            j0 = p * 2
            j1 = j0 + 1
            pltpu.make_async_copy(x_hbm.at[src_v.at[j0]], rows_v.at[0],
                                  g0).wait()
            dg1 = pltpu.async_copy(x_hbm.at[src_v.at[j1]], rows_v.at[1], g1)
            ds0 = pltpu.async_copy(rows_v.at[0], acc_sh.at[dst_v.at[j0]], s0,
                                   add=True)
            dg1.wait()
            ds0.wait()

            @pl.when(p < npairs - 1)
            def _():
                pltpu.async_copy(x_hbm.at[src_v.at[j0 + 2]], rows_v.at[0], g0)

            pltpu.async_copy(rows_v.at[1], acc_sh.at[dst_v.at[j1]], s1,
                             add=True).wait()
            return carry

        lax.fori_loop(0, npairs, _pair, 0)
    plsc.subcore_barrier()

    def _wback(t, carry):
        off = pl.multiple_of(s * ROWS_PER_TILE + t * WCHUNK, 8)
        pltpu.sync_copy(acc_sh.at[pl.ds(off, WCHUNK)],
                        part_hbm.at[c, pl.ds(off, WCHUNK)])
        return carry

    lax.fori_loop(0, ROWS_PER_TILE // WCHUNK, _wback, 0)


RBLK = 2000


def _mix_mm_tc(part_ref, init_ref, w_ref, b_ref, out_ref):
    h = (1.0 - ALPHA) * (part_ref[0] + part_ref[1]) + ALPHA * init_ref[...]
    mm = lax.dot_general(h, w_ref[...], (((1,), (1,)), ((), ())),
                         preferred_element_type=jnp.float32)
    out_ref[...] = BETA * (mm + b_ref[...]) + (1.0 - BETA) * h


def kernel(x, init_x, edge_index, W, b):
    npad = EPAD - N_EDGES
    pad_src = jnp.arange(npad, dtype=jnp.int32) % N_NODES
    src = jnp.concatenate(
        [edge_index[0], pad_src]
    ).reshape(NW, NCHUNK, CHUNK)
    pad_dst = PAD_DST + jnp.arange(npad, dtype=jnp.int32) % (NPAD - N_NODES)
    dst = jnp.concatenate(
        [edge_index[1], pad_dst]
    ).reshape(NW, NCHUNK, CHUNK)
    part = _spmm_sc(x, src, dst)
    out = pl.pallas_call(
        _mix_mm_tc,
        grid=(N_NODES // RBLK,),
        in_specs=[
            pl.BlockSpec((NC, RBLK, DIM), lambda i: (0, i, 0)),
            pl.BlockSpec((RBLK, DIM), lambda i: (i, 0)),
            pl.BlockSpec((DIM, DIM), lambda i: (0, 0)),
            pl.BlockSpec((1, DIM), lambda i: (0, 0)),
        ],
        out_specs=pl.BlockSpec((RBLK, DIM), lambda i: (i, 0)),
        out_shape=jax.ShapeDtypeStruct((N_NODES, DIM), jnp.float32),
    )(part, init_x, W, b.reshape(1, DIM))
    return out
